# baseline (speedup 1.0000x reference)
import functools

import jax
import jax.numpy as jnp
from jax import lax
from jax.experimental import pallas as pl
from jax.experimental.pallas import tpu as pltpu
from jax.experimental.pallas import tpu_sc as plsc

NUM_LAYERS = 3
NUM_BUCKETS = 65536
NUM_GRAPHS = 16
NSUB = 16
LANES = 16

K1 = -1640531535
K2 = -2048144777
K3 = -1028477379
K4 = 668265263


def _tc_argmax(x):
    n, f = x.shape

    def body(x_ref, o_ref):
        am = jnp.argmax(x_ref[...], axis=-1).astype(jnp.int32)
        o_ref[...] = am.reshape(o_ref.shape)

    br = next((b for b in (1000, 1024, 2000, 512, 400, 256, 200, 80, 8)
               if n % b == 0 and b % 8 == 0), None)
    if br is None:
        return pl.pallas_call(
            body, out_shape=jax.ShapeDtypeStruct((n,), jnp.int32))(x)
    out3 = pl.pallas_call(
        body,
        grid=(n // br,),
        in_specs=[pl.BlockSpec((br, f), lambda i: (i, 0))],
        out_specs=pl.BlockSpec((1, 1, br), lambda i: (i, 0, 0)),
        out_shape=jax.ShapeDtypeStruct((n // br, 1, br), jnp.int32),
    )(x)
    return out3.reshape(n)


def _tc_norm(counts):
    def body(c_ref, o_ref):
        h = c_ref[...].astype(jnp.float32)
        ss = jnp.sum(h * h, axis=-1, keepdims=True)
        nrm = jnp.sqrt(ss)
        o_ref[...] = h / jnp.where(nrm == 0.0, 1.0, nrm)

    return pl.pallas_call(
        body,
        out_shape=jax.ShapeDtypeStruct(counts.shape, jnp.float32),
    )(counts)


def _pick_echunk(ept, max_c=2048):
    for c in (5120, 4096, 4000, 3200, 2560, 2048, 2000, 1600, 1280, 1024, 800,
              640, 512, 400, 320, 256, 160, 128, 80, 64, 48, 32, 16):
        if c <= max_c and ept % c == 0:
            return c
    return None


def _sc_wl(colors0, ei_flat, batch_p, n_pad, e_pad):
    npt = n_pad // NSUB
    ept = e_pad // NSUB
    ech = _pick_echunk(ept)
    n_chunks = n_pad // LANES
    nkc = ept // ech
    UN = 8
    ecn = ech // LANES
    UE = next(u for u in (5, 4, 2, 1) if ecn % u == 0)
    mesh = plsc.VectorSubcoreMesh(
        core_axis_name="c", subcore_axis_name="s", num_cores=1, num_subcores=NSUB
    )

    @functools.partial(
        pl.kernel,
        out_type=jax.ShapeDtypeStruct((NUM_GRAPHS, NUM_BUCKETS), jnp.float32),
        mesh=mesh,
        scratch_types=[
            pltpu.VMEM((n_pad,), jnp.int32),
            pltpu.VMEM((n_pad,), jnp.int32),
            pltpu.VMEM((n_pad,), jnp.int32),
            pltpu.VMEM((NUM_BUCKETS,), jnp.float32),
            pltpu.VMEM((ech,), jnp.int32),
            pltpu.VMEM((ech,), jnp.int32),
            pltpu.VMEM((ech,), jnp.int32),
            pltpu.VMEM((ech,), jnp.int32),
            pltpu.VMEM((NSUB, npt), jnp.int32),
            pltpu.VMEM((npt,), jnp.int32),
            pltpu.VMEM_SHARED((NSUB, n_pad), jnp.int32),
            pltpu.VMEM_SHARED((n_pad,), jnp.int32),
            pltpu.SemaphoreType.DMA,
            pltpu.SemaphoreType.DMA,
        ],
        compiler_params=pltpu.CompilerParams(needs_layout_passes=False),
    )
    def wl_kernel(colors_hbm, ei_hbm, batch_hbm, out_hbm,
                  colors_v, h_v, neigh_v, hist_v, src_p0, src_p1,
                  dst_p0, dst_p1, part2d, cnew_st, priv_sh, colors_sh,
                  sem_a, sem_b):
        sid = lax.axis_index("s")
        sems = (sem_a, sem_b)
        srcb = (src_p0, src_p1)
        dstb = (dst_p0, dst_p1)
        zeros = jnp.zeros((LANES,), jnp.int32)
        zeros_f = jnp.zeros((LANES,), jnp.float32)
        ones_f = jnp.full((LANES,), 1.0, jnp.float32)

        pltpu.sync_copy(colors_hbm, colors_v)

        def stage_edges(layer_base, kc):
            b = layer_base + kc * ech
            s = sems[kc % 2]
            return (
                pltpu.async_copy(ei_hbm.at[pl.ds(b, ech)], srcb[kc % 2], s),
                pltpu.async_copy(ei_hbm.at[pl.ds(e_pad + b, ech)],
                                 dstb[kc % 2], s),
            )

        for _layer in range(NUM_LAYERS):
            ebase = sid * ept
            pend = stage_edges(ebase, 0)

            with jax.named_scope(f"L{_layer}_hash"):
                @plsc.parallel_loop(0, n_chunks, unroll=UN)
                def _(j):
                    i = pl.ds(j * LANES, LANES)
                    c = colors_v[i]
                    h = c * jnp.int32(K1)
                    h = h ^ jnp.int32(K2)
                    h = h ^ lax.shift_right_logical(h, 15)
                    h_v[i] = h
                    neigh_v[i] = zeros

            es = jax.named_scope(f"L{_layer}_edges")
            es.__enter__()
            for kc in range(nkc):
                if kc + 1 < nkc:
                    nxt = stage_edges(ebase, kc + 1)
                for c in pend:
                    c.wait()
                pend = nxt if kc + 1 < nkc else ()
                sb, db = srcb[kc % 2], dstb[kc % 2]

                @plsc.parallel_loop(0, ech // LANES, unroll=UE)
                def _(j):
                    i = pl.ds(j * LANES, LANES)
                    sv = sb[i]
                    dv = db[i]
                    vals = plsc.load_gather(h_v, [sv])
                    plsc.addupdate_scatter(neigh_v, [dv], vals)
            es.__exit__(None, None, None)

            rs = jax.named_scope(f"L{_layer}_reduce")
            rs.__enter__()
            pltpu.sync_copy(neigh_v, priv_sh.at[sid])
            plsc.subcore_barrier()

            nslice = pl.ds(sid * npt, npt)
            pltpu.sync_copy(priv_sh.at[pl.ds(0, NSUB), nslice], part2d)

            @plsc.parallel_loop(0, npt // LANES, unroll=4)
            def _(i):
                ii = pl.ds(i * LANES, LANES)
                nb = part2d[0, ii]
                for t in range(1, NSUB):
                    nb = nb + part2d[t, ii]
                c = colors_v[pl.ds(sid * npt + i * LANES, LANES)]
                sig = nb * jnp.int32(K3) + c * jnp.int32(K4)
                sig = sig ^ lax.shift_right_logical(sig, 13)
                cnew_st[ii] = sig & jnp.int32(0xFFFF)
            pltpu.sync_copy(cnew_st, colors_sh.at[nslice])
            plsc.subcore_barrier()
            rs.__exit__(None, None, None)
            with jax.named_scope(f"L{_layer}_bcast"):
                pltpu.sync_copy(colors_sh, colors_v)
                plsc.subcore_barrier()

        pltpu.sync_copy(batch_hbm, h_v)

        with jax.named_scope("hist_zero"):
            @plsc.parallel_loop(0, NUM_BUCKETS // LANES, unroll=UN)
            def _(j):
                hist_v[pl.ds(j * LANES, LANES)] = zeros_f

        with jax.named_scope("hist_scan"):
            @plsc.parallel_loop(0, n_chunks, unroll=UN)
            def _(j):
                i = pl.ds(j * LANES, LANES)
                plsc.addupdate_scatter(hist_v, [colors_v[i]], ones_f,
                                       mask=h_v[i] == sid)

        def ssq_body(j, accs):
            vs = [hist_v[pl.ds((j * UN + u) * LANES, LANES)] for u in range(UN)]
            return tuple(accs[u] + vs[u] * vs[u] for u in range(UN))

        with jax.named_scope("norm_ssq"):
            acc_parts = lax.fori_loop(0, NUM_BUCKETS // (LANES * UN), ssq_body,
                                      (zeros_f,) * UN)
        ssq_lanes = acc_parts[0]
        for u in range(1, UN):
            ssq_lanes = ssq_lanes + acc_parts[u]
        ssq = jnp.full((LANES,), jnp.sum(ssq_lanes), jnp.float32)
        yi = jnp.int32(0x5F3759DF) - lax.shift_right_logical(
            plsc.bitcast(ssq, jnp.int32), 1)
        y = plsc.bitcast(yi, jnp.float32)
        half = ssq * jnp.float32(0.5)
        for _ in range(4):
            y = y * (jnp.float32(1.5) - half * y * y)
        scale = jnp.where(ssq == 0.0, jnp.float32(1.0), y)

        och = NUM_BUCKETS // 8
        with jax.named_scope("norm_scale_out"):
            handles = []
            for c in range(NUM_BUCKETS // och):
                @plsc.parallel_loop(c * (och // LANES), (c + 1) * (och // LANES),
                                    unroll=UN)
                def _(j):
                    i = pl.ds(j * LANES, LANES)
                    hist_v[i] = hist_v[i] * scale

                handles.append(pltpu.async_copy(
                    hist_v.at[pl.ds(c * och, och)],
                    out_hbm.at[sid, pl.ds(c * och, och)], sems[c % 2]))
            for hdl in handles:
                hdl.wait()

    return wl_kernel(colors0, ei_flat, batch_p)


def kernel(x, edge_index, batch):
    n = x.shape[0]
    e = edge_index.shape[1]
    ei = edge_index.astype(jnp.int32)
    b32 = batch.astype(jnp.int32)

    n_pad = -(-n // (NSUB * LANES)) * (NSUB * LANES)
    colors0 = jnp.pad(_tc_argmax(x), (0, n_pad - n))
    batch_p = jnp.pad(b32, (0, n_pad - n), constant_values=NUM_GRAPHS)

    e_pad = e
    if e % NSUB != 0 or _pick_echunk(e // NSUB) is None:
        e_pad = -(-e // (NSUB * 1024)) * (NSUB * 1024)
    if e_pad != e:
        tail = jnp.stack([
            jnp.zeros((e_pad - e,), jnp.int32),
            jnp.full((e_pad - e,), n_pad - 1, jnp.int32),
        ])
        ei = jnp.concatenate([ei, tail], axis=1)

    return _sc_wl(colors0, ei.reshape(2 * e_pad), batch_p, n_pad, e_pad)


# speedup vs baseline: 1.0303x; 1.0303x over previous
import functools

import jax
import jax.numpy as jnp
from jax import lax
from jax.experimental import pallas as pl
from jax.experimental.pallas import tpu as pltpu
from jax.experimental.pallas import tpu_sc as plsc

NUM_LAYERS = 3
NUM_BUCKETS = 65536
NUM_GRAPHS = 16
NSUB = 16
LANES = 16

K1 = -1640531535
K2 = -2048144777
K3 = -1028477379
K4 = 668265263


def _tc_argmax(x):
    n, f = x.shape

    def body(x_ref, o_ref):
        am = jnp.argmax(x_ref[...], axis=-1).astype(jnp.int32)
        o_ref[...] = am.reshape(o_ref.shape)

    br = next((b for b in (2048, 2000, 1024, 1000, 512, 400, 256, 200, 80, 8)
               if n % b == 0 and b % 8 == 0), None)
    if br is None:
        return pl.pallas_call(
            body, out_shape=jax.ShapeDtypeStruct((n,), jnp.int32))(x)
    out3 = pl.pallas_call(
        body,
        grid=(n // br,),
        in_specs=[pl.BlockSpec((br, f), lambda i: (i, 0))],
        out_specs=pl.BlockSpec((1, 1, br), lambda i: (i, 0, 0)),
        out_shape=jax.ShapeDtypeStruct((n // br, 1, br), jnp.int32),
    )(x)
    return out3.reshape(n)


def _pick_echunk(ept, max_c=2048):
    for c in (5120, 4096, 4000, 3200, 2560, 2048, 2000, 1600, 1280, 1024, 800,
              640, 512, 400, 320, 256, 160, 128, 80, 64, 48, 32, 16):
        if c <= max_c and ept % c == 0:
            return c
    return None


def _sc_wl(colors0, ei_flat, batch_p, n_pad, e_pad):
    npt = n_pad // NSUB
    ept = e_pad // NSUB
    ech = _pick_echunk(ept)
    n_chunks = n_pad // LANES
    nkc = ept // ech
    UN = 8
    ecn = ech // LANES
    UE = next(u for u in (5, 4, 2, 1) if ecn % u == 0)
    mesh = plsc.VectorSubcoreMesh(
        core_axis_name="c", subcore_axis_name="s", num_cores=1, num_subcores=NSUB
    )

    @functools.partial(
        pl.kernel,
        out_type=jax.ShapeDtypeStruct((NUM_GRAPHS, NUM_BUCKETS), jnp.float32),
        mesh=mesh,
        scratch_types=[
            pltpu.VMEM((n_pad,), jnp.int32),
            pltpu.VMEM((n_pad,), jnp.int32),
            pltpu.VMEM((n_pad,), jnp.int32),
            pltpu.VMEM((NUM_BUCKETS,), jnp.float32),
            pltpu.VMEM((ech,), jnp.int32),
            pltpu.VMEM((ech,), jnp.int32),
            pltpu.VMEM((ech,), jnp.int32),
            pltpu.VMEM((ech,), jnp.int32),
            pltpu.VMEM((NSUB, npt), jnp.int32),
            pltpu.VMEM((npt,), jnp.int32),
            pltpu.VMEM_SHARED((NSUB, n_pad), jnp.int32),
            pltpu.VMEM_SHARED((n_pad,), jnp.int32),
            pltpu.SemaphoreType.DMA,
            pltpu.SemaphoreType.DMA,
        ],
        compiler_params=pltpu.CompilerParams(needs_layout_passes=False),
    )
    def wl_kernel(colors_hbm, ei_hbm, batch_hbm, out_hbm,
                  colors_v, h_v, neigh_v, hist_v, src_p0, src_p1,
                  dst_p0, dst_p1, part2d, cnew_st, priv_sh, colors_sh,
                  sem_a, sem_b):
        sid = lax.axis_index("s")
        sems = (sem_a, sem_b)
        srcb = (src_p0, src_p1)
        dstb = (dst_p0, dst_p1)
        zeros = jnp.zeros((LANES,), jnp.int32)
        zeros_f = jnp.zeros((LANES,), jnp.float32)
        ones_f = jnp.full((LANES,), 1.0, jnp.float32)

        pltpu.sync_copy(colors_hbm, colors_v)

        def stage_edges(layer_base, kc):
            b = layer_base + kc * ech
            s = sems[kc % 2]
            return (
                pltpu.async_copy(ei_hbm.at[pl.ds(b, ech)], srcb[kc % 2], s),
                pltpu.async_copy(ei_hbm.at[pl.ds(e_pad + b, ech)],
                                 dstb[kc % 2], s),
            )

        for _layer in range(NUM_LAYERS):
            ebase = sid * ept
            pend = stage_edges(ebase, 0)

            with jax.named_scope(f"L{_layer}_hash"):
                @plsc.parallel_loop(0, n_chunks, unroll=UN)
                def _(j):
                    i = pl.ds(j * LANES, LANES)
                    c = colors_v[i]
                    h = c * jnp.int32(K1)
                    h = h ^ jnp.int32(K2)
                    h = h ^ lax.shift_right_logical(h, 15)
                    h_v[i] = h
                    neigh_v[i] = zeros

            es = jax.named_scope(f"L{_layer}_edges")
            es.__enter__()
            for kc in range(nkc):
                if kc + 1 < nkc:
                    nxt = stage_edges(ebase, kc + 1)
                for c in pend:
                    c.wait()
                pend = nxt if kc + 1 < nkc else ()
                sb, db = srcb[kc % 2], dstb[kc % 2]

                @plsc.parallel_loop(0, ech // LANES, unroll=UE)
                def _(j):
                    i = pl.ds(j * LANES, LANES)
                    sv = sb[i]
                    dv = db[i]
                    vals = plsc.load_gather(h_v, [sv])
                    plsc.addupdate_scatter(neigh_v, [dv], vals)
            es.__exit__(None, None, None)

            rs = jax.named_scope(f"L{_layer}_reduce")
            rs.__enter__()
            pltpu.sync_copy(neigh_v, priv_sh.at[sid])
            plsc.subcore_barrier()

            nslice = pl.ds(sid * npt, npt)
            pltpu.sync_copy(priv_sh.at[pl.ds(0, NSUB), nslice], part2d)

            @plsc.parallel_loop(0, npt // LANES, unroll=4)
            def _(i):
                ii = pl.ds(i * LANES, LANES)
                nb = part2d[0, ii]
                for t in range(1, NSUB):
                    nb = nb + part2d[t, ii]
                c = colors_v[pl.ds(sid * npt + i * LANES, LANES)]
                sig = nb * jnp.int32(K3) + c * jnp.int32(K4)
                sig = sig ^ lax.shift_right_logical(sig, 13)
                cnew_st[ii] = sig & jnp.int32(0xFFFF)
            pltpu.sync_copy(cnew_st, colors_sh.at[nslice])
            plsc.subcore_barrier()
            rs.__exit__(None, None, None)
            with jax.named_scope(f"L{_layer}_bcast"):
                pltpu.sync_copy(colors_sh, colors_v)
                plsc.subcore_barrier()

        pltpu.sync_copy(batch_hbm, h_v)

        with jax.named_scope("hist_zero"):
            @plsc.parallel_loop(0, NUM_BUCKETS // LANES, unroll=UN)
            def _(j):
                hist_v[pl.ds(j * LANES, LANES)] = zeros_f

        with jax.named_scope("hist_scan"):
            @plsc.parallel_loop(0, n_chunks, unroll=UN)
            def _(j):
                i = pl.ds(j * LANES, LANES)
                plsc.addupdate_scatter(hist_v, [colors_v[i]], ones_f,
                                       mask=h_v[i] == sid)

        def ssq_body(j, accs):
            vs = [hist_v[pl.ds((j * UN + u) * LANES, LANES)] for u in range(UN)]
            return tuple(accs[u] + vs[u] * vs[u] for u in range(UN))

        with jax.named_scope("norm_ssq"):
            acc_parts = lax.fori_loop(0, NUM_BUCKETS // (LANES * UN), ssq_body,
                                      (zeros_f,) * UN)
        ssq_lanes = acc_parts[0]
        for u in range(1, UN):
            ssq_lanes = ssq_lanes + acc_parts[u]
        ssq = jnp.full((LANES,), jnp.sum(ssq_lanes), jnp.float32)
        yi = jnp.int32(0x5F3759DF) - lax.shift_right_logical(
            plsc.bitcast(ssq, jnp.int32), 1)
        y = plsc.bitcast(yi, jnp.float32)
        half = ssq * jnp.float32(0.5)
        for _ in range(4):
            y = y * (jnp.float32(1.5) - half * y * y)
        scale = jnp.where(ssq == 0.0, jnp.float32(1.0), y)

        och = NUM_BUCKETS // 8
        with jax.named_scope("norm_scale_out"):
            handles = []
            for c in range(NUM_BUCKETS // och):
                @plsc.parallel_loop(c * (och // LANES), (c + 1) * (och // LANES),
                                    unroll=UN)
                def _(j):
                    i = pl.ds(j * LANES, LANES)
                    hist_v[i] = hist_v[i] * scale

                handles.append(pltpu.async_copy(
                    hist_v.at[pl.ds(c * och, och)],
                    out_hbm.at[sid, pl.ds(c * och, och)], sems[c % 2]))
            for hdl in handles:
                hdl.wait()

    return wl_kernel(colors0, ei_flat, batch_p)


def kernel(x, edge_index, batch):
    n = x.shape[0]
    e = edge_index.shape[1]
    ei = edge_index.astype(jnp.int32)
    b32 = batch.astype(jnp.int32)

    n_pad = -(-n // (NSUB * LANES)) * (NSUB * LANES)
    colors0 = jnp.pad(_tc_argmax(x), (0, n_pad - n))
    batch_p = jnp.pad(b32, (0, n_pad - n), constant_values=NUM_GRAPHS)

    e_pad = e
    if e % NSUB != 0 or _pick_echunk(e // NSUB) is None:
        e_pad = -(-e // (NSUB * 1024)) * (NSUB * 1024)
    if e_pad != e:
        tail = jnp.stack([
            jnp.zeros((e_pad - e,), jnp.int32),
            jnp.full((e_pad - e,), n_pad - 1, jnp.int32),
        ])
        ei = jnp.concatenate([ei, tail], axis=1)

    return _sc_wl(colors0, ei.reshape(2 * e_pad), batch_p, n_pad, e_pad)
